# baseline (speedup 1.0000x reference)
import functools

import jax
import jax.numpy as jnp
from jax import lax
from jax.experimental import pallas as pl
from jax.experimental.pallas import tpu as pltpu
from jax.experimental.pallas import tpu_sc as plsc

VOCAB_SIZE = 1000000
EMBED_DIM = 64
BATCH = 4096
HIST_LEN = 200

_NC = 2
_NS = 16
_NW = _NC * _NS
_CHUNK = 128
_N_IDX = BATCH * HIST_LEN
_PER_W = _N_IDX // _NW
_N_CHUNKS = _PER_W // _CHUNK


def _emb_kernel(idx_hbm, table_hbm, out_hbm, idx_v, rows0, rows1, sem0, sem1):
    wid = lax.axis_index("s") * _NC + lax.axis_index("c")

    pltpu.sync_copy(idx_hbm.at[wid], idx_v)

    pltpu.async_copy(table_hbm.at[idx_v.at[0]], rows0, sem0)
    pltpu.async_copy(table_hbm.at[idx_v.at[1]], rows1, sem1)

    def body(t, carry):
        j0 = 2 * t

        def step(rows_b, sem_b, j):
            pltpu.make_async_copy(
                table_hbm.at[idx_v.at[j]], rows_b, sem_b).wait()
            pltpu.sync_copy(rows_b, out_hbm.at[wid, j])
            pltpu.async_copy(table_hbm.at[idx_v.at[j + 2]], rows_b, sem_b)

        step(rows0, sem0, j0)
        step(rows1, sem1, j0 + 1)
        return carry

    lax.fori_loop(0, _N_CHUNKS // 2 - 1, body, 0)

    j_last = _N_CHUNKS - 2
    pltpu.make_async_copy(
        table_hbm.at[idx_v.at[j_last]], rows0, sem0).wait()
    pltpu.sync_copy(rows0, out_hbm.at[wid, j_last])
    pltpu.make_async_copy(
        table_hbm.at[idx_v.at[j_last + 1]], rows1, sem1).wait()
    pltpu.sync_copy(rows1, out_hbm.at[wid, j_last + 1])


def _build():
    mesh = plsc.VectorSubcoreMesh(core_axis_name="c", subcore_axis_name="s")
    return functools.partial(
        pl.kernel,
        mesh=mesh,
        out_type=jax.ShapeDtypeStruct((_NW, _N_CHUNKS, _CHUNK, EMBED_DIM),
                                      jnp.float32),
        scratch_types=[
            pltpu.VMEM((_N_CHUNKS, _CHUNK), jnp.int32),
            pltpu.VMEM((_CHUNK, EMBED_DIM), jnp.float32),
            pltpu.VMEM((_CHUNK, EMBED_DIM), jnp.float32),
            pltpu.SemaphoreType.DMA,
            pltpu.SemaphoreType.DMA,
        ],
        compiler_params=pltpu.CompilerParams(use_tc_tiling_on_sc=False),
    )(_emb_kernel)


def kernel(x, embedding_weight):
    idx = x.reshape(_NW, _N_CHUNKS, _CHUNK).astype(jnp.int32)
    out = _build()(idx, embedding_weight)
    return out.reshape(BATCH, HIST_LEN, EMBED_DIM)
